# Initial kernel scaffold; baseline (speedup 1.0000x reference)
import jax
import jax.numpy as jnp
from jax.experimental import pallas as pl


def kernel(h, edge_index, coord, edge_attr, Wq, bq, Wkv, bkv, W1, W2):
    raise NotImplementedError("write your pallas kernel here")


# speedup vs baseline: 14.8764x; 14.8764x over previous
import numpy as np
import jax
import jax.numpy as jnp
from jax import lax
from jax.experimental import pallas as pl
from jax.experimental.pallas import tpu as pltpu
from jax.experimental.pallas import tpu_sc as plsc

N = 10000
E = 320000
D = 128
C = 4
ED = 16
H = 128
HID = 512

NP = 10240
NC, NS = 2, 16
NW = NC * NS
EPW = E // NW
CH = 80
NCH = EPW // CH
EPS = E // NS
GA = EPS // 16
GB = EPW // 16
RW = D + 16
CW = 2 * H + 16
OW = H + 16
NPN = N // NS
NSL = NP // NS

BLK1 = 1024
BE = 2000
BLK5 = 400

_f32 = jnp.float32
_NEG = -3.0e38


def _np_consts():
    sl = np.zeros((16, 48), np.float32)
    sr = np.zeros((16, 48), np.float32)
    for d in range(3):
        for i in range(4):
            for j in range(4):
                l = 16 * d + 4 * i + j
                sl[3 * i + d, l] = 1.0
                sr[3 * j + d, l] = 1.0
    sg = np.zeros((4, 16), np.float32)
    for i in range(4):
        for d in range(3):
            sg[i, 3 * i + d] = 1.0
    hot = np.zeros((1, 16), np.float32)
    hot[0, 12] = 1.0
    tmask = np.ones((1, 16), np.float32)
    tmask[0, 12:] = 0.0
    return sl, sr, sg, hot, tmask


_SL, _SR, _SG, _HOT, _TMASK = _np_consts()


def _node_body(h_ref, cp_ref, wq_ref, bq_ref, wh_ref, bkv_ref, r_ref, c_ref):
    h = h_ref[...]
    q = jnp.dot(h, wq_ref[...], preferred_element_type=_f32) + bq_ref[...]
    r_ref[...] = jnp.concatenate([q, cp_ref[...]], axis=1)
    kvn = jnp.dot(h, wh_ref[...], preferred_element_type=_f32) + bkv_ref[...]
    c_ref[...] = jnp.concatenate([kvn, cp_ref[...]], axis=1)


def _node_call(hp, cp, wq, bq2, wh, bkv2):
    return pl.pallas_call(
        _node_body,
        grid=(NP // BLK1,),
        in_specs=[
            pl.BlockSpec((BLK1, D), lambda i: (i, 0)),
            pl.BlockSpec((BLK1, 16), lambda i: (i, 0)),
            pl.BlockSpec((D, H), lambda i: (0, 0)),
            pl.BlockSpec((1, H), lambda i: (0, 0)),
            pl.BlockSpec((D, 2 * H), lambda i: (0, 0)),
            pl.BlockSpec((1, 2 * H), lambda i: (0, 0)),
        ],
        out_specs=[
            pl.BlockSpec((BLK1, RW), lambda i: (i, 0)),
            pl.BlockSpec((BLK1, CW), lambda i: (i, 0)),
        ],
        out_shape=[
            jax.ShapeDtypeStruct((NP, RW), _f32),
            jax.ShapeDtypeStruct((NP, CW), _f32),
        ],
    )(hp, cp, wq, bq2, wh, bkv2)


def _gather_body(r_hbm, c_hbm, rowi_hbm, coli_hbm, rg_hbm, cg_hbm,
                 idxr_v, idxc_v, bufr_v, bufc_v, semr, semc):
    cid = lax.axis_index("c")
    sid = lax.axis_index("s")
    w = sid * NC + cid
    pltpu.sync_copy(rowi_hbm.at[w], idxr_v)
    pltpu.sync_copy(coli_hbm.at[w], idxc_v)

    def body(j, carry):
        base = w * EPW + j * CH
        cr = pltpu.async_copy(r_hbm.at[idxr_v.at[j]], bufr_v, semr)
        cc = pltpu.async_copy(c_hbm.at[idxc_v.at[j]], bufc_v, semc)
        cr.wait()
        cc.wait()
        pltpu.sync_copy(bufr_v, rg_hbm.at[pl.ds(base, CH)])
        pltpu.sync_copy(bufc_v, cg_hbm.at[pl.ds(base, CH)])
        return carry

    lax.fori_loop(0, NCH, body, 0)


def _gather_call(r_tab, c_tab, row3, col3):
    mesh = plsc.VectorSubcoreMesh(core_axis_name="c", subcore_axis_name="s")
    fn = pl.kernel(
        _gather_body,
        out_type=[
            jax.ShapeDtypeStruct((E, RW), _f32),
            jax.ShapeDtypeStruct((E, CW), _f32),
        ],
        mesh=mesh,
        compiler_params=pltpu.CompilerParams(use_tc_tiling_on_sc=False, needs_layout_passes=False),
        scratch_types=[
            pltpu.VMEM((NCH, CH), jnp.int32),
            pltpu.VMEM((NCH, CH), jnp.int32),
            pltpu.VMEM((CH, RW), _f32),
            pltpu.VMEM((CH, CW), _f32),
            pltpu.SemaphoreType.DMA,
            pltpu.SemaphoreType.DMA,
        ],
    )
    return fn(r_tab, c_tab, row3, col3)


def _edge_body(rg_ref, cg_ref, ea_ref, wre_ref, w1_ref, w2sg_ref,
               sl_ref, sr_ref, hot_ref, out_ref, a_ref):
    rg = rg_ref[...]
    cg = cg_ref[...]
    qr = rg[:, 0:128]
    diff = rg[:, 128:144] - cg[:, 256:272]
    left = jnp.dot(diff, sl_ref[...], preferred_element_type=_f32)
    right = jnp.dot(diff, sr_ref[...], preferred_element_type=_f32)
    prod = left * right
    radial = prod[:, 0:16] + prod[:, 16:32] + prod[:, 32:48]
    re = jnp.concatenate([radial, ea_ref[...]], axis=1)
    kv = jnp.dot(re, wre_ref[...], preferred_element_type=_f32)
    k = cg[:, 0:128] + kv[:, 0:128]
    v = cg[:, 128:256] + kv[:, 128:256]
    alpha = jnp.sum(qr * k, axis=1, keepdims=True)
    u = jnp.dot(v.astype(jnp.bfloat16), w1_ref[...], preferred_element_type=_f32)
    u = u * jax.nn.sigmoid(u)
    gx = jnp.dot(u.astype(jnp.bfloat16), w2sg_ref[...], preferred_element_type=_f32)
    tail = gx * diff + alpha * hot_ref[...]
    out_ref[...] = jnp.concatenate([v, tail], axis=1)
    a_ref[...] = alpha


def _edge_call(rg, cg, ea, wre, w1b, w2sgb, sl, sr, hot):
    return pl.pallas_call(
        _edge_body,
        grid=(E // BE,),
        in_specs=[
            pl.BlockSpec((BE, RW), lambda i: (i, 0)),
            pl.BlockSpec((BE, CW), lambda i: (i, 0)),
            pl.BlockSpec((BE, ED), lambda i: (i, 0)),
            pl.BlockSpec((32, 2 * H), lambda i: (0, 0)),
            pl.BlockSpec((H, HID), lambda i: (0, 0)),
            pl.BlockSpec((HID, 16), lambda i: (0, 0)),
            pl.BlockSpec((16, 48), lambda i: (0, 0)),
            pl.BlockSpec((16, 48), lambda i: (0, 0)),
            pl.BlockSpec((1, 16), lambda i: (0, 0)),
        ],
        out_specs=[
            pl.BlockSpec((BE, OW), lambda i: (i, 0)),
            pl.BlockSpec((BE, 1), lambda i: (i, 0)),
        ],
        out_shape=[
            jax.ShapeDtypeStruct((E, OW), _f32),
            jax.ShapeDtypeStruct((E, 1), _f32),
        ],
    )(rg, cg, ea, wre, w1b, w2sgb, sl, sr, hot)


_PIB = lax.GatherScatterMode.PROMISE_IN_BOUNDS


_DNUMS = lax.GatherDimensionNumbers(
    offset_dims=(), collapsed_slice_dims=(0,), start_index_map=(0,))


def _vtake(x, i):
    return lax.gather(x, i[:, None], _DNUMS, (1,), mode=_PIB)


def _maxp_body2(rowb_hbm, ab_hbm, p_hbm,
                idxa_v, aa_v, idxb_v, ab_v, macc_v, amax_v, pout_v,
                cmb_v, cmb2_v, sh):
    cid = lax.axis_index("c")
    sid = lax.axis_index("s")
    w = sid * NC + cid
    iota = lax.iota(jnp.int32, 16)

    def initb(i, carry):
        macc_v[pl.ds(i * 16, 16)] = jnp.full((16,), _NEG, _f32)
        return carry

    lax.fori_loop(0, NP // 16, initb, 0)
    pltpu.sync_copy(rowb_hbm.at[2 * sid], idxa_v.at[pl.ds(0, GB)])
    pltpu.sync_copy(rowb_hbm.at[2 * sid + 1], idxa_v.at[pl.ds(GB, GB)])
    pltpu.sync_copy(ab_hbm.at[2 * sid], aa_v.at[pl.ds(0, GB)])
    pltpu.sync_copy(ab_hbm.at[2 * sid + 1], aa_v.at[pl.ds(GB, GB)])

    def grp(i, carry):
        ks, vs = plsc.sort_key_val(idxa_v[i, :], aa_v[i, :])
        for st in (1, 2, 4, 8):
            src = jnp.maximum(iota - st, 0)
            ok = jnp.logical_and(iota >= st, _vtake(ks, src) == ks)
            vs = jnp.maximum(vs, jnp.where(ok, _vtake(vs, src), _NEG))
        nxt = _vtake(ks, jnp.minimum(iota + 1, 15))
        last = jnp.logical_or(iota == 15, ks != nxt)
        cur = plsc.load_gather(macc_v, [ks])
        plsc.store_scatter(macc_v, [ks], jnp.maximum(cur, vs), mask=last)
        return carry

    lax.fori_loop(0, GA, grp, 0)

    pltpu.sync_copy(macc_v, sh.at[sid])
    plsc.subcore_barrier()
    base = sid * NSL
    pltpu.sync_copy(sh.at[0, pl.ds(base, NSL)], cmb_v)

    def rowred(r, carry):
        pltpu.sync_copy(sh.at[r, pl.ds(base, NSL)], cmb2_v)

        def mx(i, c2):
            sl16 = pl.ds(i * 16, 16)
            cmb_v[sl16] = jnp.maximum(cmb_v[sl16], cmb2_v[sl16])
            return c2

        lax.fori_loop(0, NSL // 16, mx, 0)
        return carry

    lax.fori_loop(1, NS, rowred, 0)
    plsc.subcore_barrier()
    pltpu.sync_copy(cmb_v, sh.at[0, pl.ds(base, NSL)])
    plsc.subcore_barrier()
    pltpu.sync_copy(sh.at[0], amax_v)

    pltpu.sync_copy(rowb_hbm.at[w], idxb_v)
    pltpu.sync_copy(ab_hbm.at[w], ab_v)

    def pb(i, carry):
        am = plsc.load_gather(amax_v, [idxb_v[i, :]])
        pout_v[i, :] = jnp.exp(ab_v[i, :] - am)
        return carry

    lax.fori_loop(0, GB, pb, 0)
    pltpu.sync_copy(pout_v, p_hbm.at[w])


def _maxp_call(rowb, alphab):
    mesh = plsc.VectorSubcoreMesh(core_axis_name="c", subcore_axis_name="s")
    fn = pl.kernel(
        _maxp_body2,
        out_type=jax.ShapeDtypeStruct((NW, GB, 16), _f32),
        mesh=mesh,
        compiler_params=pltpu.CompilerParams(use_tc_tiling_on_sc=False, needs_layout_passes=False),
        scratch_types=[
            pltpu.VMEM((GA, 16), jnp.int32),
            pltpu.VMEM((GA, 16), _f32),
            pltpu.VMEM((GB, 16), jnp.int32),
            pltpu.VMEM((GB, 16), _f32),
            pltpu.VMEM((NP,), _f32),
            pltpu.VMEM((NP,), _f32),
            pltpu.VMEM((GB, 16), _f32),
            pltpu.VMEM((NSL,), _f32),
            pltpu.VMEM((NSL,), _f32),
            pltpu.VMEM_SHARED((NS, NP), _f32),
        ],
    )
    return fn(rowb, alphab)


def _scale_body(ev_ref, p_ref, hot_ref, tm_ref, out_ref):
    ev = ev_ref[...]
    p = p_ref[...]
    tail = p * (ev[:, 128:144] * tm_ref[...] + hot_ref[...])
    out_ref[...] = jnp.concatenate([p * ev[:, 0:128], tail], axis=1)


def _scale_call(ev, p2, hot, tmask):
    return pl.pallas_call(
        _scale_body,
        grid=(E // BE,),
        in_specs=[
            pl.BlockSpec((BE, OW), lambda i: (i, 0)),
            pl.BlockSpec((BE, 1), lambda i: (i, 0)),
            pl.BlockSpec((1, 16), lambda i: (0, 0)),
            pl.BlockSpec((1, 16), lambda i: (0, 0)),
        ],
        out_specs=pl.BlockSpec((BE, OW), lambda i: (i, 0)),
        out_shape=jax.ShapeDtypeStruct((E, OW), _f32),
    )(ev, p2, hot, tmask)


def _scatter_body(pv_hbm, rowi_hbm, zz_hbm, out_hbm, idx_v, buf_v, acc_sh):
    cid = lax.axis_index("c")
    sid = lax.axis_index("s")
    w = sid * NC + cid
    pltpu.sync_copy(zz_hbm.at[pl.ds(sid * NPN, NPN)],
                    acc_sh.at[pl.ds(sid * NPN, NPN)])
    plsc.subcore_barrier()
    pltpu.sync_copy(rowi_hbm.at[w], idx_v)

    def body(j, carry):
        base = w * EPW + j * CH
        pltpu.sync_copy(pv_hbm.at[pl.ds(base, CH)], buf_v)
        pltpu.sync_copy(buf_v, acc_sh.at[idx_v.at[j]], add=True)
        return carry

    lax.fori_loop(0, NCH, body, 0)
    plsc.subcore_barrier()
    pltpu.sync_copy(acc_sh.at[pl.ds(sid * NPN, NPN)],
                    out_hbm.at[cid, pl.ds(sid * NPN, NPN)])


def _scatter_call(pv144, row3, zz):
    mesh = plsc.VectorSubcoreMesh(core_axis_name="c", subcore_axis_name="s")
    fn = pl.kernel(
        _scatter_body,
        out_type=jax.ShapeDtypeStruct((NC, N, OW), _f32),
        mesh=mesh,
        compiler_params=pltpu.CompilerParams(use_tc_tiling_on_sc=False, needs_layout_passes=False),
        scratch_types=[
            pltpu.VMEM((NCH, CH), jnp.int32),
            pltpu.VMEM((CH, OW), _f32),
            pltpu.VMEM_SHARED((N, OW), _f32),
        ],
    )
    return fn(pv144, row3, zz)


def _fin_body(p0_ref, p1_ref, h_ref, cf_ref, ho_ref, co_ref):
    acc = p0_ref[...] + p1_ref[...]
    inv = 1.0 / (acc[:, 140:141] + 1e-16)
    ho_ref[...] = h_ref[...] + acc[:, 0:128] * inv
    co_ref[...] = cf_ref[...] + jnp.clip(acc[:, 128:140] * inv, -10.0, 10.0)


def _fin_call(p0, p1, h, coordf):
    return pl.pallas_call(
        _fin_body,
        grid=(N // BLK5,),
        in_specs=[
            pl.BlockSpec((BLK5, OW), lambda i: (i, 0)),
            pl.BlockSpec((BLK5, OW), lambda i: (i, 0)),
            pl.BlockSpec((BLK5, D), lambda i: (i, 0)),
            pl.BlockSpec((BLK5, 12), lambda i: (i, 0)),
        ],
        out_specs=[
            pl.BlockSpec((BLK5, D), lambda i: (i, 0)),
            pl.BlockSpec((BLK5, 12), lambda i: (i, 0)),
        ],
        out_shape=[
            jax.ShapeDtypeStruct((N, D), _f32),
            jax.ShapeDtypeStruct((N, 12), _f32),
        ],
    )(p0, p1, h, coordf)


@jax.jit
def kernel(h, edge_index, coord, edge_attr, Wq, bq, Wkv, bkv, W1, W2):
    row = edge_index[0]
    col = edge_index[1]
    perm = np.concatenate([np.arange(0, 2 * H, 2), np.arange(1, 2 * H, 2)])
    wkv_p = Wkv[:, perm]
    bkv_p = bkv[perm]
    wre = jnp.concatenate([wkv_p[0:16], wkv_p[144:160]], axis=0)
    wh = wkv_p[16:144]
    coordf = coord.reshape(N, 3 * C)
    cp = jnp.pad(coordf, ((0, NP - N), (0, 4)))
    hp = jnp.pad(h, ((0, NP - N), (0, 0)))
    sl = jnp.asarray(_SL)
    sr = jnp.asarray(_SR)
    hot = jnp.asarray(_HOT)
    tmask = jnp.asarray(_TMASK)
    w1b = W1.astype(jnp.bfloat16)
    w2sgb = jnp.dot(W2, jnp.asarray(_SG)).astype(jnp.bfloat16)
    row3 = row.reshape(NW, NCH, CH)
    col3 = col.reshape(NW, NCH, CH)
    rowb = row.reshape(NW, GB, 16)
    zz = jnp.zeros((N, OW), _f32)

    r_tab, c_tab = _node_call(hp, cp, Wq, bq.reshape(1, H), wh,
                              bkv_p.reshape(1, 2 * H))
    rg, cg = _gather_call(r_tab, c_tab, row3, col3)
    ev144, alpha = _edge_call(rg, cg, edge_attr, wre, w1b, w2sgb, sl, sr, hot)
    alphab = alpha.reshape(NW, GB, 16)
    p = _maxp_call(rowb, alphab)
    pv144 = _scale_call(ev144, p.reshape(E, 1), hot, tmask)
    parts = _scatter_call(pv144, row3, zz)
    h_out, cof = _fin_call(parts[0], parts[1], h, coordf)
    return h_out, cof.reshape(N, C, 3)


# speedup vs baseline: 25.1602x; 1.6913x over previous
import numpy as np
import jax
import jax.numpy as jnp
from jax import lax
from jax.experimental import pallas as pl
from jax.experimental.pallas import tpu as pltpu
from jax.experimental.pallas import tpu_sc as plsc

N = 10000
E = 320000
D = 128
C = 4
ED = 16
H = 128
HID = 512

NP = 10240
NC, NS = 2, 16
NW = NC * NS
EPW = E // NW
CH = 80
NCH = EPW // CH
DPR = E * 16 // 128
NPN = N // NS
NSL = NP // NS

BLK1 = 1024
BE = 2560
BPR = BE * 16 // 128
BLK5 = 400

_f32 = jnp.float32
_NEG = -3.0e38
_PIB = lax.GatherScatterMode.PROMISE_IN_BOUNDS
_DNUMS = lax.GatherDimensionNumbers(
    offset_dims=(), collapsed_slice_dims=(0,), start_index_map=(0,))


def _vtake(x, i):
    return lax.gather(x, i[:, None], _DNUMS, (1,), mode=_PIB)


def _np_consts():
    sl = np.zeros((16, 48), np.float32)
    sr = np.zeros((16, 48), np.float32)
    for d in range(3):
        for i in range(4):
            for j in range(4):
                l = 16 * d + 4 * i + j
                sl[3 * i + d, l] = 1.0
                sr[3 * j + d, l] = 1.0
    sg = np.zeros((4, 16), np.float32)
    for i in range(4):
        for d in range(3):
            sg[i, 3 * i + d] = 1.0
    hot = np.zeros((1, 16), np.float32)
    hot[0, 12] = 1.0
    return sl, sr, sg, hot


_SL, _SR, _SG, _HOT = _np_consts()


def _node_body(h_ref, wq_ref, bq_ref, wh_ref, bkv_ref, q_ref, kn_ref, vn_ref):
    h = h_ref[...]
    q_ref[...] = jnp.dot(h, wq_ref[...], preferred_element_type=_f32) + bq_ref[...]
    kvn = jnp.dot(h, wh_ref[...], preferred_element_type=_f32) + bkv_ref[...]
    kn_ref[...] = kvn[:, 0:128]
    vn_ref[...] = kvn[:, 128:256]


def _node_call(hp, wq, bq2, wh, bkv2):
    return pl.pallas_call(
        _node_body,
        grid=(NP // BLK1,),
        in_specs=[
            pl.BlockSpec((BLK1, D), lambda i: (i, 0)),
            pl.BlockSpec((D, H), lambda i: (0, 0)),
            pl.BlockSpec((1, H), lambda i: (0, 0)),
            pl.BlockSpec((D, 2 * H), lambda i: (0, 0)),
            pl.BlockSpec((1, 2 * H), lambda i: (0, 0)),
        ],
        out_specs=[
            pl.BlockSpec((BLK1, H), lambda i: (i, 0)),
            pl.BlockSpec((BLK1, H), lambda i: (i, 0)),
            pl.BlockSpec((BLK1, H), lambda i: (i, 0)),
        ],
        out_shape=[
            jax.ShapeDtypeStruct((NP, H), _f32),
            jax.ShapeDtypeStruct((NP, H), _f32),
            jax.ShapeDtypeStruct((NP, H), _f32),
        ],
    )(hp, wq, bq2, wh, bkv2)


def _gather_body(tq_hbm, tkn_hbm, tvn_hbm, cp_hbm, rowi_hbm, coli_hbm,
                 qr_hbm, kn_hbm, vn_hbm, dp_hbm,
                 idxr_v, idxc_v, qb_v, knb_v, vnb_v, crb_v, ccb_v, db_v, sem):
    cid = lax.axis_index("c")
    sid = lax.axis_index("s")
    w = sid * NC + cid
    pltpu.sync_copy(rowi_hbm.at[w], idxr_v)
    pltpu.sync_copy(coli_hbm.at[w], idxc_v)

    def body(j, carry):
        base = w * EPW + j * CH
        c1 = pltpu.async_copy(tq_hbm.at[idxr_v.at[j]], qb_v, sem)
        c2 = pltpu.async_copy(tkn_hbm.at[idxc_v.at[j]], knb_v, sem)
        c3 = pltpu.async_copy(tvn_hbm.at[idxc_v.at[j]], vnb_v, sem)
        c4 = pltpu.async_copy(cp_hbm.at[idxr_v.at[j]], crb_v, sem)
        c5 = pltpu.async_copy(cp_hbm.at[idxc_v.at[j]], ccb_v, sem)
        c1.wait()
        c2.wait()
        c3.wait()
        c4.wait()
        c5.wait()
        for l in range(CH):
            db_v[l, :] = crb_v[l, :] - ccb_v[l, :]
        pltpu.sync_copy(qb_v, qr_hbm.at[pl.ds(base, CH)])
        pltpu.sync_copy(knb_v, kn_hbm.at[pl.ds(base, CH)])
        pltpu.sync_copy(vnb_v, vn_hbm.at[pl.ds(base, CH)])
        pltpu.sync_copy(db_v, dp_hbm.at[pl.ds(base, CH)])
        return carry

    lax.fori_loop(0, NCH, body, 0)


def _gather_call(tq, tkn, tvn, cp, row3, col3):
    mesh = plsc.VectorSubcoreMesh(core_axis_name="c", subcore_axis_name="s")
    fn = pl.kernel(
        _gather_body,
        out_type=[
            jax.ShapeDtypeStruct((E, H), _f32),
            jax.ShapeDtypeStruct((E, H), _f32),
            jax.ShapeDtypeStruct((E, H), _f32),
            jax.ShapeDtypeStruct((E, 16), _f32),
        ],
        mesh=mesh,
        compiler_params=pltpu.CompilerParams(
            use_tc_tiling_on_sc=False, needs_layout_passes=False),
        scratch_types=[
            pltpu.VMEM((NCH, CH), jnp.int32),
            pltpu.VMEM((NCH, CH), jnp.int32),
            pltpu.VMEM((CH, H), _f32),
            pltpu.VMEM((CH, H), _f32),
            pltpu.VMEM((CH, H), _f32),
            pltpu.VMEM((CH, 16), _f32),
            pltpu.VMEM((CH, 16), _f32),
            pltpu.VMEM((CH, 16), _f32),
            pltpu.SemaphoreType.DMA,
        ],
    )
    return fn(tq, tkn, tvn, cp, row3, col3)


def _edge_body(qr_ref, kn_ref, vn_ref, dp_ref, ea_ref, wre_ref, w1_ref,
               w2sg_ref, sl_ref, sr_ref, hot_ref, ev_ref, tp_ref):
    diff = dp_ref[...]
    left = jnp.dot(diff, sl_ref[...], preferred_element_type=_f32)
    right = jnp.dot(diff, sr_ref[...], preferred_element_type=_f32)
    prod = left * right
    radial = prod[:, 0:16] + prod[:, 16:32] + prod[:, 32:48]
    re = jnp.concatenate([radial, ea_ref[...]], axis=1)
    kv = jnp.dot(re, wre_ref[...], preferred_element_type=_f32)
    k = kn_ref[...] + kv[:, 0:128]
    v = vn_ref[...] + kv[:, 128:256]
    alpha = jnp.sum(qr_ref[...] * k, axis=1, keepdims=True)
    u = jnp.dot(v.astype(jnp.bfloat16), w1_ref[...], preferred_element_type=_f32)
    u = u * jax.nn.sigmoid(u)
    gx = jnp.dot(u.astype(jnp.bfloat16), w2sg_ref[...], preferred_element_type=_f32)
    tail = gx * diff + alpha * hot_ref[...]
    ev_ref[...] = v
    tp_ref[...] = tail


def _edge_call(qr, kn, vn, dp, ea, wre, w1b, w2sgb, sl, sr, hot):
    return pl.pallas_call(
        _edge_body,
        grid=(E // BE,),
        in_specs=[
            pl.BlockSpec((BE, H), lambda i: (i, 0)),
            pl.BlockSpec((BE, H), lambda i: (i, 0)),
            pl.BlockSpec((BE, H), lambda i: (i, 0)),
            pl.BlockSpec((BE, 16), lambda i: (i, 0)),
            pl.BlockSpec((BE, ED), lambda i: (i, 0)),
            pl.BlockSpec((32, 2 * H), lambda i: (0, 0)),
            pl.BlockSpec((H, HID), lambda i: (0, 0)),
            pl.BlockSpec((HID, 16), lambda i: (0, 0)),
            pl.BlockSpec((16, 48), lambda i: (0, 0)),
            pl.BlockSpec((16, 48), lambda i: (0, 0)),
            pl.BlockSpec((1, 16), lambda i: (0, 0)),
        ],
        out_specs=[
            pl.BlockSpec((BE, H), lambda i: (i, 0)),
            pl.BlockSpec((BE, 16), lambda i: (i, 0)),
        ],
        out_shape=[
            jax.ShapeDtypeStruct((E, H), _f32),
            jax.ShapeDtypeStruct((E, 16), _f32),
        ],
    )(qr, kn, vn, dp, ea, wre, w1b, w2sgb, sl, sr, hot)


ACH = 5
ANC = 2 * NCH // ACH


def _sm_body(rowi_hbm, tp_hbm, ev_hbm, zv_hbm, zt_hbm, ov_hbm, ot_hbm, mx_hbm,
             idxa_v, idxc_v, sta_v, macc_v, cmb_v, cmb2_v,
             evb_v, tpb_v, tlb_v, accv_sh, acct_sh):
    cid = lax.axis_index("c")
    sid = lax.axis_index("s")
    w = sid * NC + cid
    iota = lax.iota(jnp.int32, 16)

    pltpu.sync_copy(zv_hbm.at[pl.ds(sid * NPN, NPN)],
                    accv_sh.at[pl.ds(sid * NPN, NPN)])
    pltpu.sync_copy(zt_hbm.at[pl.ds(sid * NPN, NPN)],
                    acct_sh.at[pl.ds(sid * NPN, NPN)])

    def initb(i, carry):
        macc_v[pl.ds(i * 16, 16)] = jnp.full((16,), _NEG, _f32)
        return carry

    lax.fori_loop(0, NP // 16, initb, 0)

    def chunka(c, carry):
        wa = 2 * sid + c // (NCH // ACH)
        lr = (c % (NCH // ACH)) * ACH
        pltpu.sync_copy(rowi_hbm.at[wa, pl.ds(lr, ACH)], idxa_v)
        trow = wa * (EPW // 8) + lr * (CH // 8)
        pltpu.sync_copy(tp_hbm.at[pl.ds(trow, ACH * CH // 8)], sta_v)

        def rowa(r, carry2):
            def onegrp(k, ridx):
                a16 = plsc.load_gather(
                    sta_v, [10 * r + 2 * k + iota // 8, 16 * (iota & 7) + 12])
                ks, vs = plsc.sort_key_val(ridx, a16)
                for st in (1, 2, 4, 8):
                    src = jnp.maximum(iota - st, 0)
                    ok = jnp.logical_and(iota >= st, _vtake(ks, src) == ks)
                    vs = jnp.maximum(vs, jnp.where(ok, _vtake(vs, src), _NEG))
                nxt = _vtake(ks, jnp.minimum(iota + 1, 15))
                last = jnp.logical_or(iota == 15, ks != nxt)
                cur = plsc.load_gather(macc_v, [ks])
                plsc.store_scatter(macc_v, [ks], jnp.maximum(cur, vs), mask=last)

            for k in range(5):
                onegrp(k, idxa_v[r, pl.ds(16 * k, 16)])
            return carry2

        lax.fori_loop(0, ACH, rowa, 0)
        return carry

    lax.fori_loop(0, ANC, chunka, 0)

    pltpu.sync_copy(macc_v, mx_hbm.at[cid, sid])
    plsc.subcore_barrier()
    base = sid * NSL
    pltpu.sync_copy(mx_hbm.at[cid, 0, pl.ds(base, NSL)], cmb_v)

    def rowred(r, carry):
        pltpu.sync_copy(mx_hbm.at[cid, r, pl.ds(base, NSL)], cmb2_v)

        def mx(i, c2):
            sl16 = pl.ds(i * 16, 16)
            cmb_v[sl16] = jnp.maximum(cmb_v[sl16], cmb2_v[sl16])
            return c2

        lax.fori_loop(0, NSL // 16, mx, 0)
        return carry

    lax.fori_loop(1, NS, rowred, 0)
    plsc.subcore_barrier()
    pltpu.sync_copy(cmb_v, mx_hbm.at[cid, 0, pl.ds(base, NSL)])
    plsc.subcore_barrier()
    pltpu.sync_copy(mx_hbm.at[cid, 0], macc_v)

    hotv = jnp.where(iota == 12, 1.0, 0.0).astype(_f32)
    tmv = jnp.where(iota < 12, 1.0, 0.0).astype(_f32)

    def chunkc(j, carry):
        base_e = w * EPW + j * CH
        pltpu.sync_copy(rowi_hbm.at[w, j], idxc_v)
        pltpu.sync_copy(ev_hbm.at[pl.ds(base_e, CH)], evb_v)
        pltpu.sync_copy(tp_hbm.at[pl.ds(base_e // 8, CH // 8)], tpb_v)
        ps = []
        for k in range(5):
            idx16 = idxc_v[pl.ds(16 * k, 16)]
            a16 = plsc.load_gather(
                tpb_v, [2 * k + iota // 8, 16 * (iota & 7) + 12])
            am = plsc.load_gather(macc_v, [idx16])
            ps.append(jnp.exp(a16 - am))
        for l in range(CH):
            pv = _vtake(ps[l // 16], jnp.full((16,), l % 16, jnp.int32))
            for r8 in range(8):
                s16 = pl.ds(16 * r8, 16)
                evb_v[l, s16] = evb_v[l, s16] * pv
            t16 = tpb_v[l // 8, pl.ds(16 * (l % 8), 16)]
            tlb_v[l, :] = (t16 * tmv + hotv) * pv
        pltpu.sync_copy(evb_v, accv_sh.at[idxc_v], add=True)
        pltpu.sync_copy(tlb_v, acct_sh.at[idxc_v], add=True)
        return carry

    lax.fori_loop(0, NCH, chunkc, 0)
    plsc.subcore_barrier()
    pltpu.sync_copy(accv_sh.at[pl.ds(sid * NPN, NPN)],
                    ov_hbm.at[cid, pl.ds(sid * NPN, NPN)])
    pltpu.sync_copy(acct_sh.at[pl.ds(sid * NPN, NPN)],
                    ot_hbm.at[cid, pl.ds(sid * NPN, NPN)])


def _sm_call(row3, tp, ev, zv, zt):
    mesh = plsc.VectorSubcoreMesh(core_axis_name="c", subcore_axis_name="s")
    fn = pl.kernel(
        _sm_body,
        out_type=[
            jax.ShapeDtypeStruct((NC, N, H), _f32),
            jax.ShapeDtypeStruct((NC, N, 16), _f32),
            jax.ShapeDtypeStruct((NC, NS, NP), _f32),
        ],
        mesh=mesh,
        compiler_params=pltpu.CompilerParams(
            use_tc_tiling_on_sc=False, needs_layout_passes=False),
        scratch_types=[
            pltpu.VMEM((ACH, CH), jnp.int32),
            pltpu.VMEM((CH,), jnp.int32),
            pltpu.VMEM((ACH * CH // 8, 128), _f32),
            pltpu.VMEM((NP,), _f32),
            pltpu.VMEM((NSL,), _f32),
            pltpu.VMEM((NSL,), _f32),
            pltpu.VMEM((CH, H), _f32),
            pltpu.VMEM((CH // 8, 128), _f32),
            pltpu.VMEM((CH, 16), _f32),
            pltpu.VMEM_SHARED((N, H), _f32),
            pltpu.VMEM_SHARED((N, 16), _f32),
        ],
    )
    return fn(row3, tp, ev, zv, zt)


def _fin_body(v0_ref, v1_ref, t0_ref, t1_ref, h_ref, cf_ref, ho_ref, co_ref):
    accv = v0_ref[...] + v1_ref[...]
    acct = t0_ref[...] + t1_ref[...]
    inv = 1.0 / (acct[:, 12:13] + 1e-16)
    ho_ref[...] = h_ref[...] + accv * inv
    co_ref[...] = cf_ref[...] + jnp.clip(acct[:, 0:12] * inv, -10.0, 10.0)


def _fin_call(v0, v1, t0, t1, h, coordf):
    return pl.pallas_call(
        _fin_body,
        grid=(N // BLK5,),
        in_specs=[
            pl.BlockSpec((BLK5, H), lambda i: (i, 0)),
            pl.BlockSpec((BLK5, H), lambda i: (i, 0)),
            pl.BlockSpec((BLK5, 16), lambda i: (i, 0)),
            pl.BlockSpec((BLK5, 16), lambda i: (i, 0)),
            pl.BlockSpec((BLK5, D), lambda i: (i, 0)),
            pl.BlockSpec((BLK5, 12), lambda i: (i, 0)),
        ],
        out_specs=[
            pl.BlockSpec((BLK5, D), lambda i: (i, 0)),
            pl.BlockSpec((BLK5, 12), lambda i: (i, 0)),
        ],
        out_shape=[
            jax.ShapeDtypeStruct((N, D), _f32),
            jax.ShapeDtypeStruct((N, 12), _f32),
        ],
    )(v0, v1, t0, t1, h, coordf)


@jax.jit
def kernel(h, edge_index, coord, edge_attr, Wq, bq, Wkv, bkv, W1, W2):
    row = edge_index[0]
    col = edge_index[1]
    perm = np.concatenate([np.arange(0, 2 * H, 2), np.arange(1, 2 * H, 2)])
    wkv_p = Wkv[:, perm]
    bkv_p = bkv[perm]
    wre = jnp.concatenate([wkv_p[0:16], wkv_p[144:160]], axis=0)
    wh = wkv_p[16:144]
    coordf = coord.reshape(N, 3 * C)
    cp = jnp.pad(coordf, ((0, NP - N), (0, 4)))
    hp = jnp.pad(h, ((0, NP - N), (0, 0)))
    sl = jnp.asarray(_SL)
    sr = jnp.asarray(_SR)
    hot = jnp.asarray(_HOT)
    w1b = W1.astype(jnp.bfloat16)
    w2sgb = jnp.dot(W2, jnp.asarray(_SG)).astype(jnp.bfloat16)
    row3 = row.reshape(NW, NCH, CH)
    col3 = col.reshape(NW, NCH, CH)
    zv = jnp.zeros((N, H), _f32)
    zt = jnp.zeros((N, 16), _f32)

    tq, tkn, tvn = _node_call(hp, Wq, bq.reshape(1, H), wh,
                              bkv_p.reshape(1, 2 * H))
    qr, kn, vn, dp = _gather_call(tq, tkn, tvn, cp, row3, col3)
    ev, tp16 = _edge_call(qr, kn, vn, dp, edge_attr, wre, w1b, w2sgb, sl, sr,
                          hot)
    tp = tp16.reshape(DPR, 128)
    ov, ot, _mx = _sm_call(row3, tp, ev, zv, zt)
    h_out, cof = _fin_call(ov[0], ov[1], ot[0], ot[1], h, coordf)
    return h_out, cof.reshape(N, C, 3)


# speedup vs baseline: 29.1360x; 1.1580x over previous
import numpy as np
import jax
import jax.numpy as jnp
from jax import lax
from jax.experimental import pallas as pl
from jax.experimental.pallas import tpu as pltpu
from jax.experimental.pallas import tpu_sc as plsc

N = 10000
E = 320000
D = 128
C = 4
ED = 16
H = 128
HID = 512

NP = 10240
NC, NS = 2, 16
NW = NC * NS
EPW = E // NW
CH = 80
NCH = EPW // CH
DPR = E * 16 // 128
NPN = N // NS
NSL = NP // NS

BLK1 = 1024
BE = 2560
BPR = BE * 16 // 128
BLK5 = 400

_f32 = jnp.float32
_NEG = -3.0e38
_PIB = lax.GatherScatterMode.PROMISE_IN_BOUNDS
_DNUMS = lax.GatherDimensionNumbers(
    offset_dims=(), collapsed_slice_dims=(0,), start_index_map=(0,))


def _vtake(x, i):
    return lax.gather(x, i[:, None], _DNUMS, (1,), mode=_PIB)


def _np_consts():
    sl = np.zeros((16, 48), np.float32)
    sr = np.zeros((16, 48), np.float32)
    for d in range(3):
        for i in range(4):
            for j in range(4):
                l = 16 * d + 4 * i + j
                sl[3 * i + d, l] = 1.0
                sr[3 * j + d, l] = 1.0
    sg = np.zeros((4, 16), np.float32)
    for i in range(4):
        for d in range(3):
            sg[i, 3 * i + d] = 1.0
    hot = np.zeros((1, 16), np.float32)
    hot[0, 12] = 1.0
    return sl, sr, sg, hot


_SL, _SR, _SG, _HOT = _np_consts()


def _node_body(h_ref, wq_ref, bq_ref, wh_ref, bkv_ref, q_ref, kn_ref, vn_ref):
    h = h_ref[...]
    q_ref[...] = jnp.dot(h, wq_ref[...], preferred_element_type=_f32) + bq_ref[...]
    kvn = jnp.dot(h, wh_ref[...], preferred_element_type=_f32) + bkv_ref[...]
    kn_ref[...] = kvn[:, 0:128]
    vn_ref[...] = kvn[:, 128:256]


def _node_call(hp, wq, bq2, wh, bkv2):
    return pl.pallas_call(
        _node_body,
        grid=(NP // BLK1,),
        in_specs=[
            pl.BlockSpec((BLK1, D), lambda i: (i, 0)),
            pl.BlockSpec((D, H), lambda i: (0, 0)),
            pl.BlockSpec((1, H), lambda i: (0, 0)),
            pl.BlockSpec((D, 2 * H), lambda i: (0, 0)),
            pl.BlockSpec((1, 2 * H), lambda i: (0, 0)),
        ],
        out_specs=[
            pl.BlockSpec((BLK1, H), lambda i: (i, 0)),
            pl.BlockSpec((BLK1, H), lambda i: (i, 0)),
            pl.BlockSpec((BLK1, H), lambda i: (i, 0)),
        ],
        out_shape=[
            jax.ShapeDtypeStruct((NP, H), _f32),
            jax.ShapeDtypeStruct((NP, H), _f32),
            jax.ShapeDtypeStruct((NP, H), _f32),
        ],
    )(hp, wq, bq2, wh, bkv2)


def _gather_body(tq_hbm, tkn_hbm, tvn_hbm, cp_hbm, rowi_hbm, coli_hbm,
                 qr_hbm, kn_hbm, vn_hbm, dp_hbm,
                 idxr_v, idxc_v, qb_v, knb_v, vnb_v, crb_v, ccb_v, db_v, sem):
    cid = lax.axis_index("c")
    sid = lax.axis_index("s")
    w = sid * NC + cid
    pltpu.sync_copy(rowi_hbm.at[w], idxr_v)
    pltpu.sync_copy(coli_hbm.at[w], idxc_v)

    def body(j, carry):
        base = w * EPW + j * CH
        c1 = pltpu.async_copy(tq_hbm.at[idxr_v.at[j]], qb_v, sem)
        c2 = pltpu.async_copy(tkn_hbm.at[idxc_v.at[j]], knb_v, sem)
        c3 = pltpu.async_copy(tvn_hbm.at[idxc_v.at[j]], vnb_v, sem)
        c4 = pltpu.async_copy(cp_hbm.at[idxr_v.at[j]], crb_v, sem)
        c5 = pltpu.async_copy(cp_hbm.at[idxc_v.at[j]], ccb_v, sem)
        c1.wait()
        c2.wait()
        c3.wait()
        c4.wait()
        c5.wait()
        for l in range(CH):
            db_v[l, :] = crb_v[l, :] - ccb_v[l, :]
        pltpu.sync_copy(qb_v, qr_hbm.at[pl.ds(base, CH)])
        pltpu.sync_copy(knb_v, kn_hbm.at[pl.ds(base, CH)])
        pltpu.sync_copy(vnb_v, vn_hbm.at[pl.ds(base, CH)])
        pltpu.sync_copy(db_v, dp_hbm.at[pl.ds(base, CH)])
        return carry

    lax.fori_loop(0, NCH, body, 0)


def _gather_call(tq, tkn, tvn, cp, row3, col3):
    mesh = plsc.VectorSubcoreMesh(core_axis_name="c", subcore_axis_name="s")
    fn = pl.kernel(
        _gather_body,
        out_type=[
            jax.ShapeDtypeStruct((E, H), _f32),
            jax.ShapeDtypeStruct((E, H), _f32),
            jax.ShapeDtypeStruct((E, H), _f32),
            jax.ShapeDtypeStruct((E, 16), _f32),
        ],
        mesh=mesh,
        compiler_params=pltpu.CompilerParams(
            use_tc_tiling_on_sc=False, needs_layout_passes=False),
        scratch_types=[
            pltpu.VMEM((NCH, CH), jnp.int32),
            pltpu.VMEM((NCH, CH), jnp.int32),
            pltpu.VMEM((CH, H), _f32),
            pltpu.VMEM((CH, H), _f32),
            pltpu.VMEM((CH, H), _f32),
            pltpu.VMEM((CH, 16), _f32),
            pltpu.VMEM((CH, 16), _f32),
            pltpu.VMEM((CH, 16), _f32),
            pltpu.SemaphoreType.DMA,
        ],
    )
    return fn(tq, tkn, tvn, cp, row3, col3)


def _edge_body(qr_ref, kn_ref, vn_ref, dp_ref, eat_ref, wr_ref, wea_ref,
               w1_ref, w2sg_ref, sl_ref, sr_ref, hot_ref, ev_ref, tp_ref):
    diff = dp_ref[...]
    left = jnp.dot(diff, sl_ref[...], preferred_element_type=_f32)
    right = jnp.dot(diff, sr_ref[...], preferred_element_type=_f32)
    prod = left * right
    radial = prod[:, 0:16] + prod[:, 16:32] + prod[:, 32:48]
    kv = (jnp.dot(radial, wr_ref[...], preferred_element_type=_f32)
          + lax.dot_general(eat_ref[...], wea_ref[...],
                            (((0,), (0,)), ((), ())),
                            preferred_element_type=_f32))
    k = kn_ref[...] + kv[:, 0:128]
    v = vn_ref[...] + kv[:, 128:256]
    alpha = jnp.sum(qr_ref[...] * k, axis=1, keepdims=True)
    u = jnp.dot(v.astype(jnp.bfloat16), w1_ref[...], preferred_element_type=_f32)
    u = u * jax.nn.sigmoid(u)
    gx = jnp.dot(u.astype(jnp.bfloat16), w2sg_ref[...], preferred_element_type=_f32)
    tail = gx * diff + alpha * hot_ref[...]
    ev_ref[...] = v
    tp_ref[...] = tail


def _edge_call(qr, kn, vn, dp, eat, wr16, wea16, w1b, w2sgb, sl, sr, hot):
    return pl.pallas_call(
        _edge_body,
        grid=(E // BE,),
        in_specs=[
            pl.BlockSpec((BE, H), lambda i: (i, 0)),
            pl.BlockSpec((BE, H), lambda i: (i, 0)),
            pl.BlockSpec((BE, H), lambda i: (i, 0)),
            pl.BlockSpec((BE, 16), lambda i: (i, 0)),
            pl.BlockSpec((ED, BE), lambda i: (0, i)),
            pl.BlockSpec((16, 2 * H), lambda i: (0, 0)),
            pl.BlockSpec((16, 2 * H), lambda i: (0, 0)),
            pl.BlockSpec((H, HID), lambda i: (0, 0)),
            pl.BlockSpec((HID, 16), lambda i: (0, 0)),
            pl.BlockSpec((16, 48), lambda i: (0, 0)),
            pl.BlockSpec((16, 48), lambda i: (0, 0)),
            pl.BlockSpec((1, 16), lambda i: (0, 0)),
        ],
        out_specs=[
            pl.BlockSpec((BE, H), lambda i: (i, 0)),
            pl.BlockSpec((BE, 16), lambda i: (i, 0)),
        ],
        out_shape=[
            jax.ShapeDtypeStruct((E, H), _f32),
            jax.ShapeDtypeStruct((E, 16), _f32),
        ],
    )(qr, kn, vn, dp, eat, wr16, wea16, w1b, w2sgb, sl, sr, hot)


ACH = 5
ANC = 2 * NCH // ACH


def _sm_body(rowi_hbm, tp_hbm, ev_hbm, zv_hbm, zt_hbm, ov_hbm, ot_hbm, mx_hbm,
             idxa_v, idxc_v, idxc2_v, macc_v, cmb_v, cmb2_v,
             evb_v, evb2_v, tpb_v, tpb2_v, tlb_v, sem0, sem1,
             accv_sh, acct_sh):
    cid = lax.axis_index("c")
    sid = lax.axis_index("s")
    w = sid * NC + cid
    iota = lax.iota(jnp.int32, 16)

    pltpu.sync_copy(zv_hbm.at[pl.ds(sid * NPN, NPN)],
                    accv_sh.at[pl.ds(sid * NPN, NPN)])
    pltpu.sync_copy(zt_hbm.at[pl.ds(sid * NPN, NPN)],
                    acct_sh.at[pl.ds(sid * NPN, NPN)])

    def initb(i, carry):
        macc_v[pl.ds(i * 16, 16)] = jnp.full((16,), _NEG, _f32)
        return carry

    lax.fori_loop(0, NP // 16, initb, 0)

    def chunka(c, carry):
        wa = 2 * sid + c // (NCH // ACH)
        lr = (c % (NCH // ACH)) * ACH
        pltpu.sync_copy(rowi_hbm.at[wa, pl.ds(lr, ACH)], idxa_v)
        trow = wa * (EPW // 8) + lr * (CH // 8)
        pltpu.sync_copy(tp_hbm.at[pl.ds(trow, ACH * CH // 8)],
                        evb2_v.at[pl.ds(0, ACH * CH // 8)])

        def rowa(r, carry2):
            def onegrp(k, ridx):
                a16 = plsc.load_gather(
                    evb2_v, [10 * r + 2 * k + iota // 8, 16 * (iota & 7) + 12])
                ks, vs = plsc.sort_key_val(ridx, a16)
                for st in (1, 2, 4, 8):
                    src = jnp.maximum(iota - st, 0)
                    ok = jnp.logical_and(iota >= st, _vtake(ks, src) == ks)
                    vs = jnp.maximum(vs, jnp.where(ok, _vtake(vs, src), _NEG))
                nxt = _vtake(ks, jnp.minimum(iota + 1, 15))
                last = jnp.logical_or(iota == 15, ks != nxt)
                cur = plsc.load_gather(macc_v, [ks])
                plsc.store_scatter(macc_v, [ks], jnp.maximum(cur, vs), mask=last)

            for k in range(5):
                onegrp(k, idxa_v[r, pl.ds(16 * k, 16)])
            return carry2

        lax.fori_loop(0, ACH, rowa, 0)
        return carry

    lax.fori_loop(0, ANC, chunka, 0)

    pltpu.sync_copy(macc_v, mx_hbm.at[cid, sid])
    plsc.subcore_barrier()
    base = sid * NSL
    pltpu.sync_copy(mx_hbm.at[cid, 0, pl.ds(base, NSL)], cmb_v)

    def rowred(r, carry):
        pltpu.sync_copy(mx_hbm.at[cid, r, pl.ds(base, NSL)], cmb2_v)

        def mx(i, c2):
            sl16 = pl.ds(i * 16, 16)
            cmb_v[sl16] = jnp.maximum(cmb_v[sl16], cmb2_v[sl16])
            return c2

        lax.fori_loop(0, NSL // 16, mx, 0)
        return carry

    lax.fori_loop(1, NS, rowred, 0)
    plsc.subcore_barrier()
    pltpu.sync_copy(cmb_v, mx_hbm.at[cid, 0, pl.ds(base, NSL)])
    plsc.subcore_barrier()
    pltpu.sync_copy(mx_hbm.at[cid, 0], macc_v)

    hotv = jnp.where(iota == 12, 1.0, 0.0).astype(_f32)
    tmv = jnp.where(iota < 12, 1.0, 0.0).astype(_f32)

    def start_chunk(j, idxb, evb, tpb, sem):
        base_e = w * EPW + j * CH
        pltpu.async_copy(rowi_hbm.at[w, j], idxb, sem)
        pltpu.async_copy(ev_hbm.at[pl.ds(base_e, CH)], evb, sem)
        pltpu.async_copy(tp_hbm.at[pl.ds(base_e // 8, CH // 8)], tpb, sem)

    def proc(j, idxb, evb, tpb, sem):
        base_e = w * EPW + j * CH
        pltpu.make_async_copy(rowi_hbm.at[w, j], idxb, sem).wait()
        pltpu.make_async_copy(ev_hbm.at[pl.ds(base_e, CH)], evb, sem).wait()
        pltpu.make_async_copy(tp_hbm.at[pl.ds(base_e // 8, CH // 8)], tpb,
                              sem).wait()
        ps = []
        for k in range(5):
            idx16 = idxb[pl.ds(16 * k, 16)]
            a16 = plsc.load_gather(
                tpb, [2 * k + iota // 8, 16 * (iota & 7) + 12])
            am = plsc.load_gather(macc_v, [idx16])
            ps.append(jnp.exp(a16 - am))
        for l in range(CH):
            pv = _vtake(ps[l // 16], jnp.full((16,), l % 16, jnp.int32))
            for r8 in range(8):
                s16 = pl.ds(16 * r8, 16)
                evb[l, s16] = evb[l, s16] * pv
            t16 = tpb[l // 8, pl.ds(16 * (l % 8), 16)]
            tlb_v[l, :] = (t16 * tmv + hotv) * pv
        pltpu.sync_copy(evb, accv_sh.at[idxb], add=True)
        pltpu.sync_copy(tlb_v, acct_sh.at[idxb], add=True)

    start_chunk(0, idxc_v, evb_v, tpb_v, sem0)

    def pairloop(i, carry):
        j0 = 2 * i
        start_chunk(j0 + 1, idxc2_v, evb2_v, tpb2_v, sem1)
        proc(j0, idxc_v, evb_v, tpb_v, sem0)
        start_chunk(j0 + 2, idxc_v, evb_v, tpb_v, sem0)
        proc(j0 + 1, idxc2_v, evb2_v, tpb2_v, sem1)
        return carry

    lax.fori_loop(0, (NCH - 1) // 2, pairloop, 0)
    proc(NCH - 1, idxc_v, evb_v, tpb_v, sem0)

    plsc.subcore_barrier()
    pltpu.sync_copy(accv_sh.at[pl.ds(sid * NPN, NPN)],
                    ov_hbm.at[cid, pl.ds(sid * NPN, NPN)])
    pltpu.sync_copy(acct_sh.at[pl.ds(sid * NPN, NPN)],
                    ot_hbm.at[cid, pl.ds(sid * NPN, NPN)])


def _sm_call(row3, tp, ev, zv, zt):
    mesh = plsc.VectorSubcoreMesh(core_axis_name="c", subcore_axis_name="s")
    fn = pl.kernel(
        _sm_body,
        out_type=[
            jax.ShapeDtypeStruct((NC, N, H), _f32),
            jax.ShapeDtypeStruct((NC, N, 16), _f32),
            jax.ShapeDtypeStruct((NC, NS, NP), _f32),
        ],
        mesh=mesh,
        compiler_params=pltpu.CompilerParams(
            use_tc_tiling_on_sc=False, needs_layout_passes=False),
        scratch_types=[
            pltpu.VMEM((ACH, CH), jnp.int32),
            pltpu.VMEM((CH,), jnp.int32),
            pltpu.VMEM((CH,), jnp.int32),
            pltpu.VMEM((NP,), _f32),
            pltpu.VMEM((NSL,), _f32),
            pltpu.VMEM((NSL,), _f32),
            pltpu.VMEM((CH, H), _f32),
            pltpu.VMEM((CH, H), _f32),
            pltpu.VMEM((CH // 8, 128), _f32),
            pltpu.VMEM((CH // 8, 128), _f32),
            pltpu.VMEM((CH, 16), _f32),
            pltpu.SemaphoreType.DMA,
            pltpu.SemaphoreType.DMA,
            pltpu.VMEM_SHARED((N, H), _f32),
            pltpu.VMEM_SHARED((N, 16), _f32),
        ],
    )
    return fn(row3, tp, ev, zv, zt)


def _fin_body(v0_ref, v1_ref, t0_ref, t1_ref, h_ref, cf_ref, ho_ref, co_ref):
    accv = v0_ref[...] + v1_ref[...]
    acct = t0_ref[...] + t1_ref[...]
    inv = 1.0 / (acct[:, 12:13] + 1e-16)
    ho_ref[...] = h_ref[...] + accv * inv
    co_ref[...] = cf_ref[...] + jnp.clip(acct[:, 0:12] * inv, -10.0, 10.0)


def _fin_call(v0, v1, t0, t1, h, coordf):
    return pl.pallas_call(
        _fin_body,
        grid=(N // BLK5,),
        in_specs=[
            pl.BlockSpec((BLK5, H), lambda i: (i, 0)),
            pl.BlockSpec((BLK5, H), lambda i: (i, 0)),
            pl.BlockSpec((BLK5, 16), lambda i: (i, 0)),
            pl.BlockSpec((BLK5, 16), lambda i: (i, 0)),
            pl.BlockSpec((BLK5, D), lambda i: (i, 0)),
            pl.BlockSpec((BLK5, 12), lambda i: (i, 0)),
        ],
        out_specs=[
            pl.BlockSpec((BLK5, D), lambda i: (i, 0)),
            pl.BlockSpec((BLK5, 12), lambda i: (i, 0)),
        ],
        out_shape=[
            jax.ShapeDtypeStruct((N, D), _f32),
            jax.ShapeDtypeStruct((N, 12), _f32),
        ],
    )(v0, v1, t0, t1, h, coordf)


@jax.jit
def kernel(h, edge_index, coord, edge_attr, Wq, bq, Wkv, bkv, W1, W2):
    row = edge_index[0]
    col = edge_index[1]
    perm = np.concatenate([np.arange(0, 2 * H, 2), np.arange(1, 2 * H, 2)])
    wkv_p = Wkv[:, perm]
    bkv_p = bkv[perm]
    wr16 = wkv_p[0:16]
    wea16 = wkv_p[144:160]
    wh = wkv_p[16:144]
    coordf = coord.reshape(N, 3 * C)
    cp = jnp.pad(coordf, ((0, NP - N), (0, 4)))
    hp = jnp.pad(h, ((0, NP - N), (0, 0)))
    sl = jnp.asarray(_SL)
    sr = jnp.asarray(_SR)
    hot = jnp.asarray(_HOT)
    w1b = W1.astype(jnp.bfloat16)
    w2sgb = jnp.dot(W2, jnp.asarray(_SG)).astype(jnp.bfloat16)
    row3 = row.reshape(NW, NCH, CH)
    col3 = col.reshape(NW, NCH, CH)
    zv = jnp.zeros((N, H), _f32)
    zt = jnp.zeros((N, 16), _f32)

    tq, tkn, tvn = _node_call(hp, Wq, bq.reshape(1, H), wh,
                              bkv_p.reshape(1, 2 * H))
    qr, kn, vn, dp = _gather_call(tq, tkn, tvn, cp, row3, col3)
    ev, tp16 = _edge_call(qr, kn, vn, dp, edge_attr.T, wr16, wea16, w1b,
                          w2sgb, sl, sr, hot)
    tp = tp16.reshape(DPR, 128)
    ov, ot, _mx = _sm_call(row3, tp, ev, zv, zt)
    h_out, cof = _fin_call(ov[0], ov[1], ot[0], ot[1], h, coordf)
    return h_out, cof.reshape(N, C, 3)


# speedup vs baseline: 31.5153x; 1.0817x over previous
import numpy as np
import jax
import jax.numpy as jnp
from jax import lax
from jax.experimental import pallas as pl
from jax.experimental.pallas import tpu as pltpu
from jax.experimental.pallas import tpu_sc as plsc

N = 10000
E = 320000
D = 128
C = 4
ED = 16
H = 128
HID = 512

NP = 10240
NC, NS = 2, 16
NW = NC * NS
EPW = E // NW
CH = 80
NCH = EPW // CH
DPR = E * 16 // 128
NPN = N // NS
NSL = NP // NS

BLK1 = 1024
BE = 2560
BPR = BE * 16 // 128
BLK5 = 400

_f32 = jnp.float32
_NEG = -3.0e38
_PIB = lax.GatherScatterMode.PROMISE_IN_BOUNDS
_DNUMS = lax.GatherDimensionNumbers(
    offset_dims=(), collapsed_slice_dims=(0,), start_index_map=(0,))


def _vtake(x, i):
    return lax.gather(x, i[:, None], _DNUMS, (1,), mode=_PIB)


def _np_consts():
    sl = np.zeros((16, 48), np.float32)
    sr = np.zeros((16, 48), np.float32)
    for d in range(3):
        for i in range(4):
            for j in range(4):
                l = 16 * d + 4 * i + j
                sl[3 * i + d, l] = 1.0
                sr[3 * j + d, l] = 1.0
    sg = np.zeros((4, 16), np.float32)
    for i in range(4):
        for d in range(3):
            sg[i, 3 * i + d] = 1.0
    hot = np.zeros((1, 16), np.float32)
    hot[0, 12] = 1.0
    return sl, sr, sg, hot


_SL, _SR, _SG, _HOT = _np_consts()


def _node_body(h_ref, wq_ref, bq_ref, wh_ref, bkv_ref, q_ref, kn_ref, vn_ref):
    h = h_ref[...]
    q_ref[...] = jnp.dot(h, wq_ref[...], preferred_element_type=_f32) + bq_ref[...]
    kvn = jnp.dot(h, wh_ref[...], preferred_element_type=_f32) + bkv_ref[...]
    kn_ref[...] = kvn[:, 0:128]
    vn_ref[...] = kvn[:, 128:256]


def _node_call(hp, wq, bq2, wh, bkv2):
    return pl.pallas_call(
        _node_body,
        grid=(NP // BLK1,),
        in_specs=[
            pl.BlockSpec((BLK1, D), lambda i: (i, 0)),
            pl.BlockSpec((D, H), lambda i: (0, 0)),
            pl.BlockSpec((1, H), lambda i: (0, 0)),
            pl.BlockSpec((D, 2 * H), lambda i: (0, 0)),
            pl.BlockSpec((1, 2 * H), lambda i: (0, 0)),
        ],
        out_specs=[
            pl.BlockSpec((BLK1, H), lambda i: (i, 0)),
            pl.BlockSpec((BLK1, H), lambda i: (i, 0)),
            pl.BlockSpec((BLK1, H), lambda i: (i, 0)),
        ],
        out_shape=[
            jax.ShapeDtypeStruct((NP, H), _f32),
            jax.ShapeDtypeStruct((NP, H), _f32),
            jax.ShapeDtypeStruct((NP, H), _f32),
        ],
    )(hp, wq, bq2, wh, bkv2)


def _gather_body(tq_hbm, tkn_hbm, tvn_hbm, cp_hbm, rowi_hbm, coli_hbm,
                 qr_hbm, kn_hbm, vn_hbm, dp_hbm,
                 idxr_v, idxc_v, qb_v, qb2_v, knb_v, knb2_v, vnb_v, vnb2_v,
                 crb_v, crb2_v, ccb_v, ccb2_v, db_v, db2_v, sem0, sem1):
    cid = lax.axis_index("c")
    sid = lax.axis_index("s")
    w = sid * NC + cid
    pltpu.sync_copy(rowi_hbm.at[w], idxr_v)
    pltpu.sync_copy(coli_hbm.at[w], idxc_v)

    def start_chunk(j, qb, knb, vnb, crb, ccb, sem):
        pltpu.async_copy(tq_hbm.at[idxr_v.at[j]], qb, sem)
        pltpu.async_copy(tkn_hbm.at[idxc_v.at[j]], knb, sem)
        pltpu.async_copy(tvn_hbm.at[idxc_v.at[j]], vnb, sem)
        pltpu.async_copy(cp_hbm.at[idxr_v.at[j]], crb, sem)
        pltpu.async_copy(cp_hbm.at[idxc_v.at[j]], ccb, sem)

    def proc(j, qb, knb, vnb, crb, ccb, db, sem):
        base = w * EPW + j * CH
        pltpu.make_async_copy(tq_hbm.at[idxr_v.at[j]], qb, sem).wait()
        pltpu.make_async_copy(tkn_hbm.at[idxc_v.at[j]], knb, sem).wait()
        pltpu.make_async_copy(tvn_hbm.at[idxc_v.at[j]], vnb, sem).wait()
        pltpu.make_async_copy(cp_hbm.at[idxr_v.at[j]], crb, sem).wait()
        pltpu.make_async_copy(cp_hbm.at[idxc_v.at[j]], ccb, sem).wait()
        for l in range(CH):
            db[l, :] = crb[l, :] - ccb[l, :]
        pltpu.sync_copy(qb, qr_hbm.at[pl.ds(base, CH)])
        pltpu.sync_copy(knb, kn_hbm.at[pl.ds(base, CH)])
        pltpu.sync_copy(vnb, vn_hbm.at[pl.ds(base, CH)])
        pltpu.sync_copy(db, dp_hbm.at[pl.ds(base, CH)])

    start_chunk(0, qb_v, knb_v, vnb_v, crb_v, ccb_v, sem0)

    def pairloop(i, carry):
        j0 = 2 * i
        start_chunk(j0 + 1, qb2_v, knb2_v, vnb2_v, crb2_v, ccb2_v, sem1)
        proc(j0, qb_v, knb_v, vnb_v, crb_v, ccb_v, db_v, sem0)
        start_chunk(j0 + 2, qb_v, knb_v, vnb_v, crb_v, ccb_v, sem0)
        proc(j0 + 1, qb2_v, knb2_v, vnb2_v, crb2_v, ccb2_v, db2_v, sem1)
        return carry

    lax.fori_loop(0, (NCH - 1) // 2, pairloop, 0)
    proc(NCH - 1, qb_v, knb_v, vnb_v, crb_v, ccb_v, db_v, sem0)


def _gather_call(tq, tkn, tvn, cp, row3, col3):
    mesh = plsc.VectorSubcoreMesh(core_axis_name="c", subcore_axis_name="s")
    fn = pl.kernel(
        _gather_body,
        out_type=[
            jax.ShapeDtypeStruct((E, H), _f32),
            jax.ShapeDtypeStruct((E, H), _f32),
            jax.ShapeDtypeStruct((E, H), _f32),
            jax.ShapeDtypeStruct((E, 16), _f32),
        ],
        mesh=mesh,
        compiler_params=pltpu.CompilerParams(
            use_tc_tiling_on_sc=False, needs_layout_passes=False),
        scratch_types=[
            pltpu.VMEM((NCH, CH), jnp.int32),
            pltpu.VMEM((NCH, CH), jnp.int32),
            pltpu.VMEM((CH, H), _f32),
            pltpu.VMEM((CH, H), _f32),
            pltpu.VMEM((CH, H), _f32),
            pltpu.VMEM((CH, H), _f32),
            pltpu.VMEM((CH, H), _f32),
            pltpu.VMEM((CH, H), _f32),
            pltpu.VMEM((CH, 16), _f32),
            pltpu.VMEM((CH, 16), _f32),
            pltpu.VMEM((CH, 16), _f32),
            pltpu.VMEM((CH, 16), _f32),
            pltpu.VMEM((CH, 16), _f32),
            pltpu.VMEM((CH, 16), _f32),
            pltpu.SemaphoreType.DMA,
            pltpu.SemaphoreType.DMA,
        ],
    )
    return fn(tq, tkn, tvn, cp, row3, col3)


def _edge_body(qr_ref, kn_ref, vn_ref, dp_ref, eat_ref, wr_ref, wea_ref,
               w1_ref, w2sg_ref, sl_ref, sr_ref, hot_ref, ev_ref, tp_ref):
    diff = dp_ref[...]
    left = jnp.dot(diff, sl_ref[...], preferred_element_type=_f32)
    right = jnp.dot(diff, sr_ref[...], preferred_element_type=_f32)
    prod = left * right
    radial = prod[:, 0:16] + prod[:, 16:32] + prod[:, 32:48]
    kv = (jnp.dot(radial, wr_ref[...], preferred_element_type=_f32)
          + lax.dot_general(eat_ref[...], wea_ref[...],
                            (((0,), (0,)), ((), ())),
                            preferred_element_type=_f32))
    k = kn_ref[...] + kv[:, 0:128]
    v = vn_ref[...] + kv[:, 128:256]
    alpha = jnp.sum(qr_ref[...] * k, axis=1, keepdims=True)
    u = jnp.dot(v.astype(jnp.bfloat16), w1_ref[...], preferred_element_type=_f32)
    u = u * jax.nn.sigmoid(u)
    gx = jnp.dot(u.astype(jnp.bfloat16), w2sg_ref[...], preferred_element_type=_f32)
    tail = gx * diff + alpha * hot_ref[...]
    ev_ref[...] = v
    tp_ref[...] = tail


def _edge_call(qr, kn, vn, dp, eat, wr16, wea16, w1b, w2sgb, sl, sr, hot):
    return pl.pallas_call(
        _edge_body,
        grid=(E // BE,),
        in_specs=[
            pl.BlockSpec((BE, H), lambda i: (i, 0)),
            pl.BlockSpec((BE, H), lambda i: (i, 0)),
            pl.BlockSpec((BE, H), lambda i: (i, 0)),
            pl.BlockSpec((BE, 16), lambda i: (i, 0)),
            pl.BlockSpec((ED, BE), lambda i: (0, i)),
            pl.BlockSpec((16, 2 * H), lambda i: (0, 0)),
            pl.BlockSpec((16, 2 * H), lambda i: (0, 0)),
            pl.BlockSpec((H, HID), lambda i: (0, 0)),
            pl.BlockSpec((HID, 16), lambda i: (0, 0)),
            pl.BlockSpec((16, 48), lambda i: (0, 0)),
            pl.BlockSpec((16, 48), lambda i: (0, 0)),
            pl.BlockSpec((1, 16), lambda i: (0, 0)),
        ],
        out_specs=[
            pl.BlockSpec((BE, H), lambda i: (i, 0)),
            pl.BlockSpec((BE, 16), lambda i: (i, 0)),
        ],
        out_shape=[
            jax.ShapeDtypeStruct((E, H), _f32),
            jax.ShapeDtypeStruct((E, 16), _f32),
        ],
    )(qr, kn, vn, dp, eat, wr16, wea16, w1b, w2sgb, sl, sr, hot)


ACH = 5
ANC = 2 * NCH // ACH


def _sm_body(rowi_hbm, tp_hbm, ev_hbm, zv_hbm, zt_hbm, ov_hbm, ot_hbm, mx_hbm,
             idxa_v, idxc_v, idxc2_v, macc_v, cmb_v, cmb2_v,
             evb_v, evb2_v, tpb_v, tpb2_v, tlb_v, sem0, sem1,
             accv_sh, acct_sh):
    cid = lax.axis_index("c")
    sid = lax.axis_index("s")
    w = sid * NC + cid
    iota = lax.iota(jnp.int32, 16)

    pltpu.sync_copy(zv_hbm.at[pl.ds(sid * NPN, NPN)],
                    accv_sh.at[pl.ds(sid * NPN, NPN)])
    pltpu.sync_copy(zt_hbm.at[pl.ds(sid * NPN, NPN)],
                    acct_sh.at[pl.ds(sid * NPN, NPN)])

    def initb(i, carry):
        macc_v[pl.ds(i * 16, 16)] = jnp.full((16,), _NEG, _f32)
        return carry

    lax.fori_loop(0, NP // 16, initb, 0)

    def chunka(c, carry):
        wa = 2 * sid + c // (NCH // ACH)
        lr = (c % (NCH // ACH)) * ACH
        pltpu.sync_copy(rowi_hbm.at[wa, pl.ds(lr, ACH)], idxa_v)
        trow = wa * (EPW // 8) + lr * (CH // 8)
        pltpu.sync_copy(tp_hbm.at[pl.ds(trow, ACH * CH // 8)],
                        evb2_v.at[pl.ds(0, ACH * CH // 8)])

        def rowa(r, carry2):
            def onegrp(k, ridx):
                a16 = plsc.load_gather(
                    evb2_v, [10 * r + 2 * k + iota // 8, 16 * (iota & 7) + 12])
                ks, vs = plsc.sort_key_val(ridx, a16)
                for st in (1, 2, 4, 8):
                    src = jnp.maximum(iota - st, 0)
                    ok = jnp.logical_and(iota >= st, _vtake(ks, src) == ks)
                    vs = jnp.maximum(vs, jnp.where(ok, _vtake(vs, src), _NEG))
                nxt = _vtake(ks, jnp.minimum(iota + 1, 15))
                last = jnp.logical_or(iota == 15, ks != nxt)
                cur = plsc.load_gather(macc_v, [ks])
                plsc.store_scatter(macc_v, [ks], jnp.maximum(cur, vs), mask=last)

            for k in range(5):
                onegrp(k, idxa_v[r, pl.ds(16 * k, 16)])
            return carry2

        lax.fori_loop(0, ACH, rowa, 0)
        return carry

    lax.fori_loop(0, ANC, chunka, 0)

    pltpu.sync_copy(macc_v, mx_hbm.at[cid, sid])
    plsc.subcore_barrier()
    base = sid * NSL
    pltpu.sync_copy(mx_hbm.at[cid, 0, pl.ds(base, NSL)], cmb_v)

    def rowred(r, carry):
        pltpu.sync_copy(mx_hbm.at[cid, r, pl.ds(base, NSL)], cmb2_v)

        def mx(i, c2):
            sl16 = pl.ds(i * 16, 16)
            cmb_v[sl16] = jnp.maximum(cmb_v[sl16], cmb2_v[sl16])
            return c2

        lax.fori_loop(0, NSL // 16, mx, 0)
        return carry

    lax.fori_loop(1, NS, rowred, 0)
    plsc.subcore_barrier()
    pltpu.sync_copy(cmb_v, mx_hbm.at[cid, 0, pl.ds(base, NSL)])
    plsc.subcore_barrier()
    pltpu.sync_copy(mx_hbm.at[cid, 0], macc_v)

    hotv = jnp.where(iota == 12, 1.0, 0.0).astype(_f32)
    tmv = jnp.where(iota < 12, 1.0, 0.0).astype(_f32)

    def start_chunk(j, idxb, evb, tpb, sem):
        base_e = w * EPW + j * CH
        pltpu.async_copy(rowi_hbm.at[w, j], idxb, sem)
        pltpu.async_copy(ev_hbm.at[pl.ds(base_e, CH)], evb, sem)
        pltpu.async_copy(tp_hbm.at[pl.ds(base_e // 8, CH // 8)], tpb, sem)

    def proc(j, idxb, evb, tpb, sem):
        base_e = w * EPW + j * CH
        pltpu.make_async_copy(rowi_hbm.at[w, j], idxb, sem).wait()
        pltpu.make_async_copy(ev_hbm.at[pl.ds(base_e, CH)], evb, sem).wait()
        pltpu.make_async_copy(tp_hbm.at[pl.ds(base_e // 8, CH // 8)], tpb,
                              sem).wait()
        ps = []
        for k in range(5):
            idx16 = idxb[pl.ds(16 * k, 16)]
            a16 = plsc.load_gather(
                tpb, [2 * k + iota // 8, 16 * (iota & 7) + 12])
            am = plsc.load_gather(macc_v, [idx16])
            ps.append(jnp.exp(a16 - am))
        for l in range(CH):
            pv = _vtake(ps[l // 16], jnp.full((16,), l % 16, jnp.int32))
            for r8 in range(8):
                s16 = pl.ds(16 * r8, 16)
                evb[l, s16] = evb[l, s16] * pv
            t16 = tpb[l // 8, pl.ds(16 * (l % 8), 16)]
            tlb_v[l, :] = (t16 * tmv + hotv) * pv
        pltpu.sync_copy(evb, accv_sh.at[idxb], add=True)
        pltpu.sync_copy(tlb_v, acct_sh.at[idxb], add=True)

    start_chunk(0, idxc_v, evb_v, tpb_v, sem0)

    def pairloop(i, carry):
        j0 = 2 * i
        start_chunk(j0 + 1, idxc2_v, evb2_v, tpb2_v, sem1)
        proc(j0, idxc_v, evb_v, tpb_v, sem0)
        start_chunk(j0 + 2, idxc_v, evb_v, tpb_v, sem0)
        proc(j0 + 1, idxc2_v, evb2_v, tpb2_v, sem1)
        return carry

    lax.fori_loop(0, (NCH - 1) // 2, pairloop, 0)
    proc(NCH - 1, idxc_v, evb_v, tpb_v, sem0)

    plsc.subcore_barrier()
    pltpu.sync_copy(accv_sh.at[pl.ds(sid * NPN, NPN)],
                    ov_hbm.at[cid, pl.ds(sid * NPN, NPN)])
    pltpu.sync_copy(acct_sh.at[pl.ds(sid * NPN, NPN)],
                    ot_hbm.at[cid, pl.ds(sid * NPN, NPN)])


def _sm_call(row3, tp, ev, zv, zt):
    mesh = plsc.VectorSubcoreMesh(core_axis_name="c", subcore_axis_name="s")
    fn = pl.kernel(
        _sm_body,
        out_type=[
            jax.ShapeDtypeStruct((NC, N, H), _f32),
            jax.ShapeDtypeStruct((NC, N, 16), _f32),
            jax.ShapeDtypeStruct((NC, NS, NP), _f32),
        ],
        mesh=mesh,
        compiler_params=pltpu.CompilerParams(
            use_tc_tiling_on_sc=False, needs_layout_passes=False),
        scratch_types=[
            pltpu.VMEM((ACH, CH), jnp.int32),
            pltpu.VMEM((CH,), jnp.int32),
            pltpu.VMEM((CH,), jnp.int32),
            pltpu.VMEM((NP,), _f32),
            pltpu.VMEM((NSL,), _f32),
            pltpu.VMEM((NSL,), _f32),
            pltpu.VMEM((CH, H), _f32),
            pltpu.VMEM((CH, H), _f32),
            pltpu.VMEM((CH // 8, 128), _f32),
            pltpu.VMEM((CH // 8, 128), _f32),
            pltpu.VMEM((CH, 16), _f32),
            pltpu.SemaphoreType.DMA,
            pltpu.SemaphoreType.DMA,
            pltpu.VMEM_SHARED((N, H), _f32),
            pltpu.VMEM_SHARED((N, 16), _f32),
        ],
    )
    return fn(row3, tp, ev, zv, zt)


def _fin_body(v0_ref, v1_ref, t0_ref, t1_ref, h_ref, cf_ref, ho_ref, co_ref):
    accv = v0_ref[...] + v1_ref[...]
    acct = t0_ref[...] + t1_ref[...]
    inv = 1.0 / (acct[:, 12:13] + 1e-16)
    ho_ref[...] = h_ref[...] + accv * inv
    co_ref[...] = cf_ref[...] + jnp.clip(acct[:, 0:12] * inv, -10.0, 10.0)


def _fin_call(v0, v1, t0, t1, h, coordf):
    return pl.pallas_call(
        _fin_body,
        grid=(N // BLK5,),
        in_specs=[
            pl.BlockSpec((BLK5, H), lambda i: (i, 0)),
            pl.BlockSpec((BLK5, H), lambda i: (i, 0)),
            pl.BlockSpec((BLK5, 16), lambda i: (i, 0)),
            pl.BlockSpec((BLK5, 16), lambda i: (i, 0)),
            pl.BlockSpec((BLK5, D), lambda i: (i, 0)),
            pl.BlockSpec((BLK5, 12), lambda i: (i, 0)),
        ],
        out_specs=[
            pl.BlockSpec((BLK5, D), lambda i: (i, 0)),
            pl.BlockSpec((BLK5, 12), lambda i: (i, 0)),
        ],
        out_shape=[
            jax.ShapeDtypeStruct((N, D), _f32),
            jax.ShapeDtypeStruct((N, 12), _f32),
        ],
    )(v0, v1, t0, t1, h, coordf)


@jax.jit
def kernel(h, edge_index, coord, edge_attr, Wq, bq, Wkv, bkv, W1, W2):
    row = edge_index[0]
    col = edge_index[1]
    perm = np.concatenate([np.arange(0, 2 * H, 2), np.arange(1, 2 * H, 2)])
    wkv_p = Wkv[:, perm]
    bkv_p = bkv[perm]
    wr16 = wkv_p[0:16]
    wea16 = wkv_p[144:160]
    wh = wkv_p[16:144]
    coordf = coord.reshape(N, 3 * C)
    cp = jnp.pad(coordf, ((0, NP - N), (0, 4)))
    hp = jnp.pad(h, ((0, NP - N), (0, 0)))
    sl = jnp.asarray(_SL)
    sr = jnp.asarray(_SR)
    hot = jnp.asarray(_HOT)
    w1b = W1.astype(jnp.bfloat16)
    w2sgb = jnp.dot(W2, jnp.asarray(_SG)).astype(jnp.bfloat16)
    row3 = row.reshape(NW, NCH, CH)
    col3 = col.reshape(NW, NCH, CH)
    zv = jnp.zeros((N, H), _f32)
    zt = jnp.zeros((N, 16), _f32)

    tq, tkn, tvn = _node_call(hp, Wq, bq.reshape(1, H), wh,
                              bkv_p.reshape(1, 2 * H))
    qr, kn, vn, dp = _gather_call(tq, tkn, tvn, cp, row3, col3)
    ev, tp16 = _edge_call(qr, kn, vn, dp, edge_attr.T, wr16, wea16, w1b,
                          w2sgb, sl, sr, hot)
    tp = tp16.reshape(DPR, 128)
    ov, ot, _mx = _sm_call(row3, tp, ev, zv, zt)
    h_out, cof = _fin_call(ov[0], ov[1], ot[0], ot[1], h, coordf)
    return h_out, cof.reshape(N, C, 3)
